# baseline (speedup 1.0000x reference)
import jax
import jax.numpy as jnp
from jax import lax
from jax.experimental import pallas as pl
from jax.experimental.pallas import tpu as pltpu
from jax.experimental.pallas import tpu_sc as plsc

VOCAB = 100000
EMBED_DIM = 64
BATCH = 4096
SEQ = 50
NUM_INDICES = BATCH * SEQ

_info = plsc.get_sparse_core_info()
NC, NS = _info.num_cores, _info.num_subcores
NW = NC * NS
BPW = BATCH // NW
PER_W = BPW * SEQ
NCHUNK = SEQ


def _embed_kernel(idx_hbm, table_hbm, out_hbm,
                  idx_all, idx2, rows, obuf, g_sems, w_sems):
    wid = lax.axis_index("s") * NC + lax.axis_index("c")
    base = wid * PER_W
    bbase = wid * BPW

    pltpu.sync_copy(idx_hbm.at[pl.ds(base, PER_W)], idx_all)

    iota = lax.iota(jnp.int32, 16)
    iota_seq = iota * SEQ

    def _reorder(s, carry):
        for blg in range(BPW // 16):
            src = plsc.load_gather(idx_all, [iota_seq + (blg * 16 * SEQ + s)])
            idx2[pl.ds(s * BPW + blg * 16, 16)] = src
        return carry
    lax.fori_loop(0, SEQ, _reorder, 0)

    drows = [iota + d0 for d0 in range(0, EMBED_DIM, 16)]

    gathers = [None] * NCHUNK
    writes = [None] * NCHUNK

    def _fire_gather(c):
        gathers[c] = pltpu.async_copy(
            table_hbm.at[idx2.at[pl.ds(c * BPW, BPW)]],
            rows.at[c % 2], g_sems.at[c % 2])

    _fire_gather(0)
    for c in range(NCHUNK):
        b = c % 2
        gathers[c].wait()
        if c + 1 < NCHUNK:
            _fire_gather(c + 1)
        if c >= 2:
            writes[c - 2].wait()

        @plsc.parallel_loop(0, BPW, unroll=8)
        def _tr(bl):
            colv = jnp.full((16,), bl, dtype=jnp.int32)
            for j in range(EMBED_DIM // 16):
                v = plsc.load_gather(rows.at[b], [colv, drows[j]])
                plsc.store_scatter(obuf.at[b], [drows[j], colv], v)

        writes[c] = pltpu.async_copy(
            obuf.at[b], out_hbm.at[c, :, pl.ds(bbase, BPW)], w_sems.at[b])

    writes[NCHUNK - 2].wait()
    writes[NCHUNK - 1].wait()


@jax.jit
def _embed(idx_flat, weight):
    mesh = plsc.VectorSubcoreMesh(core_axis_name="c", subcore_axis_name="s")
    return pl.kernel(
        _embed_kernel,
        out_type=jax.ShapeDtypeStruct((SEQ, EMBED_DIM, BATCH), jnp.float32),
        mesh=mesh,
        scratch_types=[
            pltpu.VMEM((PER_W,), jnp.int32),
            pltpu.VMEM((PER_W,), jnp.int32),
            pltpu.VMEM((2, BPW, EMBED_DIM), jnp.float32),
            pltpu.VMEM((2, EMBED_DIM, BPW), jnp.float32),
            pltpu.SemaphoreType.DMA((2,)),
            pltpu.SemaphoreType.DMA((2,)),
        ],
        compiler_params=pltpu.CompilerParams(use_tc_tiling_on_sc=False,
                                             needs_layout_passes=False),
    )(idx_flat, weight)


def kernel(input_sentence, weight):
    idx_flat = input_sentence.reshape(-1).astype(jnp.int32)
    out_t = _embed(idx_flat, weight)
    return jnp.transpose(out_t, (2, 0, 1))


# speedup vs baseline: 1.3205x; 1.3205x over previous
import jax
import jax.numpy as jnp
from jax import lax
from jax.experimental import pallas as pl
from jax.experimental.pallas import tpu as pltpu
from jax.experimental.pallas import tpu_sc as plsc

VOCAB = 100000
EMBED_DIM = 64
NUM_INDICES = 4096 * 50

_info = plsc.get_sparse_core_info()
NC, NS = _info.num_cores, _info.num_subcores
NW = NC * NS
PER_W = NUM_INDICES // NW
CHUNK = 640
NCHUNK = PER_W // CHUNK
NBUF = 3


def _embed_kernel(idx_hbm, table_hbm, out_hbm, idx_all, rows, g_sems, w_sems):
    wid = lax.axis_index("s") * NC + lax.axis_index("c")
    base = wid * PER_W

    pltpu.sync_copy(idx_hbm.at[pl.ds(base, PER_W)], idx_all)

    gathers = [None] * NCHUNK
    writes = [None] * NCHUNK
    for g in range(NCHUNK):
        b = g % NBUF
        if g >= NBUF:
            writes[g - NBUF].wait()
        gathers[g] = pltpu.async_copy(
            table_hbm.at[idx_all.at[pl.ds(g * CHUNK, CHUNK)]],
            rows.at[b], g_sems.at[b])
        if g >= 1:
            pb = (g - 1) % NBUF
            gathers[g - 1].wait()
            writes[g - 1] = pltpu.async_copy(
                rows.at[pb], out_hbm.at[pl.ds(base + (g - 1) * CHUNK, CHUNK)],
                w_sems.at[pb])
    gathers[NCHUNK - 1].wait()
    lb = (NCHUNK - 1) % NBUF
    writes[NCHUNK - 1] = pltpu.async_copy(
        rows.at[lb], out_hbm.at[pl.ds(base + (NCHUNK - 1) * CHUNK, CHUNK)],
        w_sems.at[lb])
    for g in range(NCHUNK - NBUF, NCHUNK):
        if g >= 0:
            writes[g].wait()


@jax.jit
def _embed(idx_flat, weight):
    mesh = plsc.VectorSubcoreMesh(core_axis_name="c", subcore_axis_name="s")
    return pl.kernel(
        _embed_kernel,
        out_type=jax.ShapeDtypeStruct((NUM_INDICES, EMBED_DIM), jnp.float32),
        mesh=mesh,
        scratch_types=[
            pltpu.VMEM((PER_W,), jnp.int32),
            pltpu.VMEM((NBUF, CHUNK, EMBED_DIM), jnp.float32),
            pltpu.SemaphoreType.DMA((NBUF,)),
            pltpu.SemaphoreType.DMA((NBUF,)),
        ],
        compiler_params=pltpu.CompilerParams(use_tc_tiling_on_sc=False),
    )(idx_flat, weight)


def kernel(input_sentence, weight):
    B, S = input_sentence.shape
    idx_flat = input_sentence.reshape(-1).astype(jnp.int32)
    out = _embed(idx_flat, weight)
    return out.reshape(B, S, EMBED_DIM)
